# Initial kernel scaffold; baseline (speedup 1.0000x reference)
import jax
import jax.numpy as jnp
from jax.experimental import pallas as pl


def kernel(idxs, emb_table, ln_gamma, ln_beta, W1, b1, W2, b2):
    raise NotImplementedError("write your pallas kernel here")


# speedup vs baseline: 6.3251x; 6.3251x over previous
import functools

import jax
import jax.numpy as jnp
from jax.experimental import pallas as pl

L = 8192
D = 128
VOCAB = 21


def _fused_kernel(idx_ref, tab_ref, gamma_ref, beta_ref, w1_ref, b1_ref,
                  w2_ref, b2_ref, out_ref):
    ids = idx_ref[:]

    def body(v, pooled):
        cnt = jnp.sum(jnp.where(ids == v, 1.0, 0.0))
        row = tab_ref[pl.ds(v, 1), :]
        return pooled + cnt * row

    pooled = jax.lax.fori_loop(
        0, VOCAB, body, jnp.zeros((1, D), jnp.float32)) * (1.0 / L)

    mu = jnp.mean(pooled)
    var = jnp.mean((pooled - mu) ** 2)
    xn = (pooled - mu) * jax.lax.rsqrt(var + 1e-5)
    xn = xn * gamma_ref[:] + beta_ref[:]

    h = jnp.dot(xn, w1_ref[:], preferred_element_type=jnp.float32) + b1_ref[:]
    h = 0.5 * h * (1.0 + jax.lax.erf(h * (2.0 ** -0.5)))
    out = jnp.dot(h, w2_ref[:], preferred_element_type=jnp.float32) + b2_ref[:]
    out_ref[:] = out


@functools.partial(jax.jit, static_argnums=())
def kernel(idxs, emb_table, ln_gamma, ln_beta, W1, b1, W2, b2):
    ids2d = idxs.astype(jnp.int32).reshape(L // D, D)
    out = pl.pallas_call(
        _fused_kernel,
        out_shape=jax.ShapeDtypeStruct((1, D), jnp.float32),
    )(ids2d, emb_table, ln_gamma.reshape(1, D), ln_beta.reshape(1, D),
      W1, b1.reshape(1, 2 * D), W2, b2.reshape(1, D))
    return out.reshape(D)
